# Initial kernel scaffold; baseline (speedup 1.0000x reference)
import jax
import jax.numpy as jnp
from jax.experimental import pallas as pl


def kernel(z1_features, z2_features, labels):
    raise NotImplementedError("write your pallas kernel here")


# speedup vs baseline: 10.5110x; 10.5110x over previous
import jax
import jax.numpy as jnp
from jax.experimental import pallas as pl
from jax.experimental.pallas import tpu as pltpu

TAU_ = 1.0
C_ = 100
CPAD_ = 128


def _loss_kernel(z1_ref, z2_ref, lab_ref, out_ref):
    f32 = jnp.float32
    lab = lab_ref[...]
    b = lab.shape[0]

    col = jax.lax.broadcasted_iota(jnp.int32, (b, CPAD_), 1)
    oh = (lab == col).astype(f32)

    def normalize(z):
        ss = jnp.sum(z * z, axis=1, keepdims=True)
        inv = 1.0 / jnp.maximum(jnp.sqrt(ss), 1e-12)
        return z * inv

    n1 = normalize(z1_ref[...])
    n2 = normalize(z2_ref[...])

    dn_rows = (((0,), (0,)), ((), ()))
    sums = (jax.lax.dot_general(oh, n1, dn_rows, preferred_element_type=f32)
            + jax.lax.dot_general(oh, n2, dn_rows, preferred_element_type=f32))
    counts = 2.0 * jnp.sum(oh, axis=0, keepdims=True)
    inv_cnt = 1.0 / jnp.maximum(counts, 1.0)

    vmask = (jax.lax.broadcasted_iota(jnp.int32, (1, CPAD_), 1) < C_).astype(f32)

    dn_feat = (((1,), (1,)), ((), ()))

    def view_loss(n):
        sim = jax.lax.dot_general(n, sums, dn_feat,
                                  preferred_element_type=f32) * inv_cnt
        sim = sim * (1.0 / TAU_)
        p = jnp.sum(sim * oh, axis=1, keepdims=True)
        s = jnp.sum(jnp.exp(sim - p) * vmask, axis=1, keepdims=True)
        return jnp.log(s) - p

    total = jnp.sum(view_loss(n1) + view_loss(n2), axis=0, keepdims=True)
    out_ref[...] = total * (1.0 / (2.0 * b))


def kernel(z1_features, z2_features, labels):
    b = labels.shape[0]
    lab2d = labels.astype(jnp.int32).reshape(b, 1)
    out = pl.pallas_call(
        _loss_kernel,
        out_shape=jax.ShapeDtypeStruct((1, 1), jnp.float32),
        compiler_params=pltpu.CompilerParams(
            vmem_limit_bytes=100 * 1024 * 1024,
        ),
    )(z1_features, z2_features, lab2d)
    return out[0, 0]


# speedup vs baseline: 10.9461x; 1.0414x over previous
import jax
import jax.numpy as jnp
from jax.experimental import pallas as pl
from jax.experimental.pallas import tpu as pltpu

TAU_ = 1.0
C_ = 100
CPAD_ = 128


def _loss_kernel(z1_ref, z2_ref, lab_ref, out_ref):
    f32 = jnp.float32
    bf16 = jnp.bfloat16
    lab = lab_ref[...]
    b = lab.shape[0]

    col = jax.lax.broadcasted_iota(jnp.int32, (b, CPAD_), 1)
    oh = (lab == col).astype(f32)

    def prep(z):
        ss = jnp.sum(z * z, axis=1, keepdims=True)
        inv = jax.lax.rsqrt(jnp.maximum(ss, 1e-24))
        return z.astype(bf16), inv

    zb1, inv1 = prep(z1_ref[...])
    zb2, inv2 = prep(z2_ref[...])

    ohs1 = (oh * inv1).astype(bf16)
    ohs2 = (oh * inv2).astype(bf16)
    dn_rows = (((0,), (0,)), ((), ()))
    sums = (jax.lax.dot_general(ohs1, zb1, dn_rows, preferred_element_type=f32)
            + jax.lax.dot_general(ohs2, zb2, dn_rows, preferred_element_type=f32))
    sumsb = sums.astype(bf16)

    counts = 2.0 * jnp.sum(oh, axis=0, keepdims=True)
    inv_cnt = 1.0 / jnp.maximum(counts, 1.0)

    vmask = (jax.lax.broadcasted_iota(jnp.int32, (1, CPAD_), 1) < C_).astype(f32)

    dn_feat = (((1,), (1,)), ((), ()))

    def view_loss(zb, inv):
        simr = jax.lax.dot_general(zb, sumsb, dn_feat,
                                   preferred_element_type=f32)
        sim = simr * inv_cnt * inv * (1.0 / TAU_)
        p = jnp.sum(sim * oh, axis=1, keepdims=True)
        s = jnp.sum(jnp.exp(sim - p) * vmask, axis=1, keepdims=True)
        return jnp.log(s) - p

    total = jnp.sum(view_loss(zb1, inv1) + view_loss(zb2, inv2),
                    axis=0, keepdims=True)
    out_ref[...] = total * (1.0 / (2.0 * b))


def kernel(z1_features, z2_features, labels):
    b = labels.shape[0]
    lab2d = labels.astype(jnp.int32).reshape(b, 1)
    out = pl.pallas_call(
        _loss_kernel,
        out_shape=jax.ShapeDtypeStruct((1, 1), jnp.float32),
        compiler_params=pltpu.CompilerParams(
            vmem_limit_bytes=100 * 1024 * 1024,
        ),
    )(z1_features, z2_features, lab2d)
    return out[0, 0]
